# baseline (speedup 1.0000x reference)
import functools

import jax
import jax.numpy as jnp
from jax import lax
from jax.experimental import pallas as pl
from jax.experimental.pallas import tpu as pltpu
from jax.experimental.pallas import tpu_sc as plsc

VOCAB = 1000000
DIM = 64
MAXLEN = 200
BATCH = 1024

TOKENS = BATCH * MAXLEN
NW = 32
PER_W = TOKENS // NW
CHUNK = 128
GATHERS = CHUNK // 128
STEPS = PER_W // CHUNK


def _emb_kernel(x_hbm, tok_hbm, pos_hbm, out_hbm,
                xraw_v, idx2_v, rows_v, out_v, pos_v,
                sg0, sg1, so0, so1):
    c = lax.axis_index("c")
    s = lax.axis_index("s")
    wid = s * 2 + c

    pltpu.sync_copy(pos_hbm, pos_v)

    sgs = (sg0, sg1)
    sos = (so0, so1)

    def load_idx(j, b):
        pltpu.sync_copy(x_hbm.at[wid * STEPS + j], xraw_v.at[b])
        for g in range(GATHERS):
            for m in range(128 // 16):
                sl = pl.ds(m * 16, 16)
                idx2_v[b, g, sl] = lax.shift_right_logical(xraw_v[b, g, sl], 1)

    def fire_gathers(j, b):
        for g in range(GATHERS):
            pltpu.async_copy(
                tok_hbm.at[idx2_v.at[b, g]],
                rows_v.at[pl.ds((b * GATHERS + g) * 128, 128)],
                sgs[b])

    def wait_gathers(j, b):
        for g in range(GATHERS):
            pltpu.make_async_copy(
                tok_hbm.at[idx2_v.at[b, g]],
                rows_v.at[pl.ds((b * GATHERS + g) * 128, 128)],
                sgs[b]).wait()

    def out_dma_refs(j, b):
        base = wid * PER_W + j * CHUNK
        return out_v.at[pl.ds(b * CHUNK, CHUNK)], out_hbm.at[pl.ds(base, CHUNK)]

    load_idx(0, 0)
    fire_gathers(0, 0)
    load_idx(1, 1)
    fire_gathers(1, 1)

    def step_body(j, b):
        off = lax.rem(j * CHUNK, MAXLEN)
        wait_gathers(j, b)

        @pl.when(j >= 2)
        def _():
            src, dst = out_dma_refs(j - 2, b)
            pltpu.make_async_copy(src, dst, sos[b]).wait()

        def add_pos(g16, _):
            for blk in range(GATHERS):
                hv = (xraw_v[b, blk, pl.ds(g16 * 16, 16)] & 1) * DIM
                for l in range(16):
                    row = blk * 128 + g16 * 16 + l
                    h = hv[l]
                    p = lax.rem(off + row, MAXLEN)
                    for dd in range(DIM // 16):
                        sl = pl.ds(dd * 16, 16)
                        out_v[b * CHUNK + row, sl] = (
                            rows_v[(b * GATHERS + blk) * 128 + g16 * 16 + l,
                                   pl.ds(h + dd * 16, 16)]
                            + pos_v[p, sl])
            return 0

        lax.fori_loop(0, 128 // 16, add_pos, 0, unroll=4)

        @pl.when(j + 2 < STEPS)
        def _():
            load_idx(j + 2, b)
            fire_gathers(j + 2, b)

        src, dst = out_dma_refs(j, b)
        pltpu.async_copy(src, dst, sos[b])

    def step(j, _):
        for b in range(2):
            @pl.when(lax.rem(j, 2) == b)
            def _(b=b):
                step_body(j, b)
        return 0

    lax.fori_loop(0, STEPS, step, 0)

    for jj in (STEPS - 2, STEPS - 1):
        src, dst = out_dma_refs(jj, jj % 2)
        pltpu.make_async_copy(src, dst, sos[jj % 2]).wait()


def kernel(x, token_table, pos_table):
    xf = x.reshape(NW * STEPS, GATHERS, 128).astype(jnp.int32)
    tok2 = token_table.reshape(VOCAB // 2, 2 * DIM)
    mesh = plsc.VectorSubcoreMesh(core_axis_name="c", subcore_axis_name="s")
    run = functools.partial(
        pl.kernel,
        mesh=mesh,
        out_type=jax.ShapeDtypeStruct((TOKENS, DIM), jnp.float32),
        scratch_types=[
            pltpu.VMEM((2, GATHERS, 128), jnp.int32),
            pltpu.VMEM((2, GATHERS, 128), jnp.int32),
            pltpu.VMEM((2 * CHUNK, 2 * DIM), jnp.float32),
            pltpu.VMEM((2 * CHUNK, DIM), jnp.float32),
            pltpu.VMEM((MAXLEN, DIM), jnp.float32),
            pltpu.SemaphoreType.DMA,
            pltpu.SemaphoreType.DMA,
            pltpu.SemaphoreType.DMA,
            pltpu.SemaphoreType.DMA,
        ],
    )(_emb_kernel)
    out = run(xf, tok2, pos_table)
    return out.reshape(BATCH, MAXLEN, DIM)
